# Initial kernel scaffold; baseline (speedup 1.0000x reference)
import functools

import jax
import jax.numpy as jnp
from jax import lax
from jax.experimental import pallas as pl
from jax.experimental.pallas import tpu as pltpu
from jax.experimental.pallas import tpu_sc as plsc

N_NODES = 10000
N_EDGES = 320000
D_FEAT = 128
NC, NS, L = 2, 16, 16
NW = NC * NS
NPAD = 10240
ROWS = 80
EPT = ROWS * 128
EPAD = NW * EPT
NPS = NPAD // NS
DEAD0 = 10200

_MESH = plsc.VectorSubcoreMesh(core_axis_name="c", subcore_axis_name="s")


def _rsqrt16(d):
    i = plsc.bitcast(d, jnp.int32)
    i = jnp.int32(0x5F3759DF) - lax.shift_right_logical(i, jnp.int32(1))
    y = plsc.bitcast(i, jnp.float32)
    for _ in range(3):
        y = y * (1.5 - 0.5 * d * y * y)
    return y


def _mv_body(x_ref, w_ref, o_ref):
    o_ref[...] = jnp.sum(x_ref[...] * w_ref[...], axis=1, keepdims=True)


def _fin_body(a_ref, b_ref, d_ref, bias_ref, o_ref):
    h = d_ref[...] * (a_ref[...] + b_ref[...]) + bias_ref[...]
    o_ref[...] = h * h


@functools.partial(
    pl.kernel,
    mesh=_MESH,
    out_type=[jax.ShapeDtypeStruct((NC, NPAD), jnp.float32)],
    scratch_types=[
        pltpu.VMEM((ROWS, 128), jnp.int32),
        pltpu.VMEM((ROWS, 128), jnp.float32),
        pltpu.VMEM((NPAD,), jnp.float32),
        pltpu.VMEM((NPS,), jnp.float32),
        pltpu.VMEM_SHARED((NPAD,), jnp.float32),
    ],
)
def _k_deg(dst3, ones2, zeros1, degp, dst_v, vals_v, zb, slice_v, z_sh):
    cid = lax.axis_index("c")
    sid = lax.axis_index("s")
    wid = sid * NC + cid
    pltpu.sync_copy(dst3.at[wid], dst_v)
    pltpu.sync_copy(ones2, vals_v)

    @pl.when(sid == 0)
    def _():
        pltpu.sync_copy(zeros1, zb)
        pltpu.sync_copy(zb, z_sh)

    plsc.subcore_barrier()
    pltpu.sync_copy(vals_v, z_sh.at[dst_v], add=True)
    plsc.subcore_barrier()
    s0 = pl.multiple_of(sid * NPS, NPS)
    pltpu.sync_copy(z_sh.at[pl.ds(s0, NPS)], slice_v)
    pltpu.sync_copy(slice_v, degp.at[cid, pl.ds(s0, NPS)])


def _hop_body(first_hop, src3, dst3, parts, aux, zeros1, *rest):
    if first_hop:
        (tp_out, dis_out, src_v, dst_v, vals_v,
         pa, pb, yv, uv, u_full, zb, u_sh, z_sh) = rest
    else:
        (tp_out, src_v, dst_v, vals_v,
         pa, pb, yv, uv, u_full, zb, u_sh, z_sh) = rest
    cid = lax.axis_index("c")
    sid = lax.axis_index("s")
    wid = sid * NC + cid
    s0 = pl.multiple_of(sid * NPS, NPS)

    pltpu.sync_copy(parts.at[0, pl.ds(s0, NPS)], pa)
    pltpu.sync_copy(parts.at[1, pl.ds(s0, NPS)], pb)
    pltpu.sync_copy(aux.at[pl.ds(s0, NPS)], yv)
    for k in range(NPS // L):
        sl = pl.ds(k * L, L)
        a = pa[sl]
        b_ = pb[sl]
        c_ = yv[sl]
        if first_hop:
            dis = _rsqrt16(a + b_ + 1.0)
            uv[sl] = dis * c_
            yv[sl] = dis
        else:
            uv[sl] = c_ * c_ * (a + b_)
    pltpu.sync_copy(uv, u_sh.at[pl.ds(s0, NPS)])
    if first_hop:
        @pl.when(cid == 0)
        def _():
            pltpu.sync_copy(yv, dis_out.at[pl.ds(s0, NPS)])

    pltpu.sync_copy(src3.at[wid], src_v)
    pltpu.sync_copy(dst3.at[wid], dst_v)
    plsc.subcore_barrier()

    pltpu.sync_copy(u_sh, u_full)

    @pl.when(jnp.logical_and(sid == 0, cid == 0))
    def _():
        pltpu.sync_copy(u_full, z_sh)

    @pl.when(jnp.logical_and(sid == 0, cid == 1))
    def _():
        pltpu.sync_copy(zeros1, zb)
        pltpu.sync_copy(zb, z_sh)

    plsc.subcore_barrier()

    def fill(j, carry):
        for c in range(128 // L):
            sl = pl.ds(c * L, L)
            vals_v[j, sl] = plsc.load_gather(u_full, [src_v[j, sl]])
        return carry

    lax.fori_loop(0, ROWS, fill, 0)

    pltpu.sync_copy(vals_v, z_sh.at[dst_v], add=True)
    plsc.subcore_barrier()

    pltpu.sync_copy(z_sh.at[pl.ds(s0, NPS)], uv)
    pltpu.sync_copy(uv, tp_out.at[cid, pl.ds(s0, NPS)])


_HOP_SCRATCH = [
    pltpu.VMEM((ROWS, 128), jnp.int32),
    pltpu.VMEM((ROWS, 128), jnp.int32),
    pltpu.VMEM((ROWS, 128), jnp.float32),
    pltpu.VMEM((NPS,), jnp.float32),
    pltpu.VMEM((NPS,), jnp.float32),
    pltpu.VMEM((NPS,), jnp.float32),
    pltpu.VMEM((NPS,), jnp.float32),
    pltpu.VMEM((NPAD,), jnp.float32),
    pltpu.VMEM((NPAD,), jnp.float32),
    pltpu.VMEM_SHARED((NPAD,), jnp.float32),
    pltpu.VMEM_SHARED((NPAD,), jnp.float32),
]

_k_hop1 = functools.partial(
    pl.kernel,
    mesh=_MESH,
    out_type=[jax.ShapeDtypeStruct((NC, NPAD), jnp.float32),
              jax.ShapeDtypeStruct((NPAD,), jnp.float32)],
    scratch_types=list(_HOP_SCRATCH),
)(functools.partial(_hop_body, True))

_k_hop2 = functools.partial(
    pl.kernel,
    mesh=_MESH,
    out_type=[jax.ShapeDtypeStruct((NC, NPAD), jnp.float32)],
    scratch_types=list(_HOP_SCRATCH),
)(functools.partial(_hop_body, False))


def kernel(x, edge_index, W, b):
    x = x.astype(jnp.float32)
    src = edge_index[0].astype(jnp.int32)
    dst = edge_index[1].astype(jnp.int32)

    n_pad_e = EPAD - N_EDGES
    pad_src = jnp.full((n_pad_e,), DEAD0, dtype=jnp.int32)
    pad_dst = DEAD0 + (jnp.arange(n_pad_e, dtype=jnp.int32) % (NPAD - DEAD0))
    src3 = jnp.concatenate([src, pad_src]).reshape(NW, ROWS, 128)
    dst3 = jnp.concatenate([dst, pad_dst]).reshape(NW, ROWS, 128)

    xpad = jnp.pad(x, ((0, NPAD - N_NODES), (0, 0)))
    wrow = W.astype(jnp.float32).reshape(1, D_FEAT)
    ones2 = jnp.ones((ROWS, 128), jnp.float32)
    zeros1 = jnp.zeros((NPAD,), jnp.float32)

    y = pl.pallas_call(
        _mv_body,
        out_shape=jax.ShapeDtypeStruct((NPAD, 1), jnp.float32),
    )(xpad, wrow)
    y1 = y.reshape(NPAD)

    degp = _k_deg(dst3, ones2, zeros1)
    if isinstance(degp, (tuple, list)):
        (degp,) = degp
    t1p, dis = _k_hop1(src3, dst3, degp, y1, zeros1)
    t2p = _k_hop2(src3, dst3, t1p, dis, zeros1)
    if isinstance(t2p, (tuple, list)):
        (t2p,) = t2p

    a2 = t2p.reshape(NC, ROWS, 128)
    d2 = dis.reshape(ROWS, 128)
    bias2 = jnp.broadcast_to(b.astype(jnp.float32).reshape(1, 1), (ROWS, 128))
    o2 = pl.pallas_call(
        _fin_body,
        out_shape=jax.ShapeDtypeStruct((ROWS, 128), jnp.float32),
    )(a2[0], a2[1], d2, bias2)
    return o2.reshape(NPAD, 1)[:N_NODES]


# speedup vs baseline: 92.2399x; 92.2399x over previous
import functools

import jax
import jax.numpy as jnp
from jax import lax
from jax.experimental import pallas as pl
from jax.experimental.pallas import tpu as pltpu
from jax.experimental.pallas import tpu_sc as plsc

N_NODES = 10000
N_EDGES = 320000
D_FEAT = 128
NC, NS, L = 2, 16, 16
NW = NC * NS
NPAD = 10240
ROWS = 80
EPT = ROWS * 128
EPAD = NW * EPT
NPS = NPAD // NS
DEAD0 = 10200

def _mesh():
    return plsc.VectorSubcoreMesh(core_axis_name="c", subcore_axis_name="s")


def _mv_body(x_ref, w_ref, da_ref, db_ref, y_ref, dis_ref):
    y_ref[...] = jnp.sum(x_ref[...] * w_ref[...], axis=1, keepdims=True)
    deg = da_ref[...] + db_ref[...] + 1.0
    dis_ref[...] = 1.0 / jnp.sqrt(deg)


def _fin_body(a_ref, b_ref, d_ref, bias_ref, o_ref):
    h = d_ref[...] * (a_ref[...] + b_ref[...]) + bias_ref[...]
    o_ref[...] = h * h


def _k_deg_body(dst3, ones2, zeros1, degp, dst_v, vals_v, zb, slice_v, z_sh):
    cid = lax.axis_index("c")
    sid = lax.axis_index("s")
    wid = sid * NC + cid
    pltpu.sync_copy(dst3.at[wid], dst_v)
    pltpu.sync_copy(ones2, vals_v)

    @pl.when(sid == 0)
    def _():
        pltpu.sync_copy(zeros1, zb)
        pltpu.sync_copy(zb, z_sh)

    plsc.subcore_barrier()

    def scat(j, carry):
        pltpu.sync_copy(vals_v.at[j], z_sh.at[dst_v.at[j]], add=True)
        return carry

    lax.fori_loop(0, ROWS, scat, 0)
    plsc.subcore_barrier()
    s0 = pl.multiple_of(sid * NPS, NPS)
    pltpu.sync_copy(z_sh.at[pl.ds(s0, NPS)], slice_v)
    pltpu.sync_copy(slice_v, degp.at[cid, pl.ds(s0, NPS)])


def _hop_body(first_hop, src3, dst3, parts, aux, zeros1, *rest):
    (tp_out, src_v, dst_v, vals_v,
     pa, pb, yv, uv, u_full, zb, u_sh, z_sh) = rest
    cid = lax.axis_index("c")
    sid = lax.axis_index("s")
    wid = sid * NC + cid
    s0 = pl.multiple_of(sid * NPS, NPS)

    pltpu.sync_copy(parts.at[0, pl.ds(s0, NPS)], pa)
    pltpu.sync_copy(parts.at[1, pl.ds(s0, NPS)], pb)
    pltpu.sync_copy(aux.at[pl.ds(s0, NPS)], yv)
    for k in range(NPS // L):
        sl = pl.ds(k * L, L)
        a = pa[sl]
        b_ = pb[sl]
        if first_hop:
            uv[sl] = a * b_
        else:
            c_ = yv[sl]
            uv[sl] = c_ * c_ * (a + b_)
    pltpu.sync_copy(uv, u_sh.at[pl.ds(s0, NPS)])

    pltpu.sync_copy(src3.at[wid], src_v)
    pltpu.sync_copy(dst3.at[wid], dst_v)
    plsc.subcore_barrier()

    pltpu.sync_copy(u_sh, u_full)

    @pl.when(jnp.logical_and(sid == 0, cid == 0))
    def _():
        pltpu.sync_copy(u_full, z_sh)

    @pl.when(jnp.logical_and(sid == 0, cid == 1))
    def _():
        pltpu.sync_copy(zeros1, zb)
        pltpu.sync_copy(zb, z_sh)

    plsc.subcore_barrier()

    def fill(j, carry):
        for c in range(128 // L):
            sl = pl.ds(c * L, L)
            vals_v[j, sl] = plsc.load_gather(u_full, [src_v[j, sl]])
        pltpu.sync_copy(vals_v.at[j], z_sh.at[dst_v.at[j]], add=True)
        return carry

    lax.fori_loop(0, ROWS, fill, 0)
    plsc.subcore_barrier()

    pltpu.sync_copy(z_sh.at[pl.ds(s0, NPS)], uv)
    pltpu.sync_copy(uv, tp_out.at[cid, pl.ds(s0, NPS)])


def _hop_scratch():
    return [
        pltpu.VMEM((ROWS, 128), jnp.int32),
        pltpu.VMEM((ROWS, 128), jnp.int32),
        pltpu.VMEM((ROWS, 128), jnp.float32),
        pltpu.VMEM((NPS,), jnp.float32),
        pltpu.VMEM((NPS,), jnp.float32),
        pltpu.VMEM((NPS,), jnp.float32),
        pltpu.VMEM((NPS,), jnp.float32),
        pltpu.VMEM((NPAD,), jnp.float32),
        pltpu.VMEM((NPAD,), jnp.float32),
        pltpu.VMEM_SHARED((NPAD,), jnp.float32),
        pltpu.VMEM_SHARED((NPAD,), jnp.float32),
    ]


@functools.lru_cache(maxsize=None)
def _sc_kernels():
    params = pltpu.CompilerParams(needs_layout_passes=False)
    k_deg = functools.partial(
        pl.kernel,
        mesh=_mesh(),
        compiler_params=params,
        out_type=[jax.ShapeDtypeStruct((NC, NPAD), jnp.float32)],
        scratch_types=[
            pltpu.VMEM((ROWS, 128), jnp.int32),
            pltpu.VMEM((ROWS, 128), jnp.float32),
            pltpu.VMEM((NPAD,), jnp.float32),
            pltpu.VMEM((NPS,), jnp.float32),
            pltpu.VMEM_SHARED((NPAD,), jnp.float32),
        ],
    )(_k_deg_body)
    k_hop1 = functools.partial(
        pl.kernel,
        mesh=_mesh(),
        compiler_params=params,
        out_type=[jax.ShapeDtypeStruct((NC, NPAD), jnp.float32)],
        scratch_types=_hop_scratch(),
    )(functools.partial(_hop_body, True))
    k_hop2 = functools.partial(
        pl.kernel,
        mesh=_mesh(),
        compiler_params=params,
        out_type=[jax.ShapeDtypeStruct((NC, NPAD), jnp.float32)],
        scratch_types=_hop_scratch(),
    )(functools.partial(_hop_body, False))
    return k_deg, k_hop1, k_hop2


def kernel(x, edge_index, W, b):
    x = x.astype(jnp.float32)
    src = edge_index[0].astype(jnp.int32)
    dst = edge_index[1].astype(jnp.int32)

    n_pad_e = EPAD - N_EDGES
    pad_src = jnp.full((n_pad_e,), DEAD0, dtype=jnp.int32)
    pad_dst = DEAD0 + (jnp.arange(n_pad_e, dtype=jnp.int32) % (NPAD - DEAD0))
    src3 = jnp.concatenate([src, pad_src]).reshape(NW, ROWS, 128)
    dst3 = jnp.concatenate([dst, pad_dst]).reshape(NW, ROWS, 128)

    xpad = jnp.pad(x, ((0, NPAD - N_NODES), (0, 0)))
    wrow = W.astype(jnp.float32).reshape(1, D_FEAT)
    ones2 = jnp.ones((ROWS, 128), jnp.float32)
    zeros1 = jnp.zeros((NPAD,), jnp.float32)

    _k_deg, _k_hop1, _k_hop2 = _sc_kernels()
    degp = _k_deg(dst3, ones2, zeros1)
    if isinstance(degp, (tuple, list)):
        (degp,) = degp

    dp = degp.reshape(NC, ROWS, 128)
    y, d2 = pl.pallas_call(
        _mv_body,
        out_shape=[jax.ShapeDtypeStruct((NPAD, 1), jnp.float32),
                   jax.ShapeDtypeStruct((ROWS, 128), jnp.float32)],
    )(xpad, wrow, dp[0], dp[1])
    y1 = y.reshape(NPAD)
    dis = d2.reshape(NPAD)

    dy2 = jnp.stack([dis, y1])
    t1p = _k_hop1(src3, dst3, dy2, y1, zeros1)
    if isinstance(t1p, (tuple, list)):
        (t1p,) = t1p
    t2p = _k_hop2(src3, dst3, t1p, dis, zeros1)
    if isinstance(t2p, (tuple, list)):
        (t2p,) = t2p

    a2 = t2p.reshape(NC, ROWS, 128)
    bias2 = jnp.broadcast_to(b.astype(jnp.float32).reshape(1, 1), (ROWS, 128))
    o2 = pl.pallas_call(
        _fin_body,
        out_shape=jax.ShapeDtypeStruct((ROWS, 128), jnp.float32),
    )(a2[0], a2[1], d2, bias2)
    return o2.reshape(NPAD, 1)[:N_NODES]


# speedup vs baseline: 111.0826x; 1.2043x over previous
import functools

import jax
import jax.numpy as jnp
from jax import lax
from jax.experimental import pallas as pl
from jax.experimental.pallas import tpu as pltpu
from jax.experimental.pallas import tpu_sc as plsc

N_NODES = 10000
N_EDGES = 320000
D_FEAT = 128
NC, NS, L = 2, 16, 16
NW = NC * NS
NPAD = 10240
ROWS = 80
EPT = ROWS * 128
EPAD = NW * EPT
NPS = NPAD // NS
DEAD0 = 10200

def _mesh():
    return plsc.VectorSubcoreMesh(core_axis_name="c", subcore_axis_name="s")


def _mv_body(x_ref, w_ref, da_ref, db_ref, y_ref, dis_ref):
    y_ref[...] = jnp.sum(x_ref[...] * w_ref[...], axis=1, keepdims=True)
    deg = da_ref[...] + db_ref[...] + 1.0
    dis_ref[...] = 1.0 / jnp.sqrt(deg)


def _fin_body(a_ref, b_ref, d_ref, bias_ref, o_ref):
    h = d_ref[...] * (a_ref[...] + b_ref[...]) + bias_ref[...]
    o_ref[...] = h * h


def _k_deg_body(dst3, ones2, zeros1, degp, dst_v, vals_v, zb, slice_v, z_sh,
                sem):
    cid = lax.axis_index("c")
    sid = lax.axis_index("s")
    wid = sid * NC + cid
    pltpu.sync_copy(dst3.at[wid], dst_v)
    pltpu.sync_copy(ones2, vals_v)

    @pl.when(sid == 0)
    def _():
        pltpu.sync_copy(zeros1, zb)
        pltpu.sync_copy(zb, z_sh)

    plsc.subcore_barrier()

    def scat(j, carry):
        pltpu.async_copy(vals_v.at[j], z_sh.at[dst_v.at[j]], sem, add=True)
        return carry

    lax.fori_loop(0, ROWS, scat, 0)

    def drain(j, carry):
        pltpu.make_async_copy(vals_v.at[j], z_sh.at[dst_v.at[j]], sem).wait()
        return carry

    lax.fori_loop(0, ROWS, drain, 0)
    plsc.subcore_barrier()
    s0 = pl.multiple_of(sid * NPS, NPS)
    pltpu.sync_copy(z_sh.at[pl.ds(s0, NPS)], slice_v)
    pltpu.sync_copy(slice_v, degp.at[cid, pl.ds(s0, NPS)])


def _hop_body(first_hop, src3, dst3, parts, aux, zeros1, *rest):
    (tp_out, src_v, dst_v, vals_v,
     pa, pb, yv, uv, u_full, zb, u_sh, z_sh, sem) = rest
    cid = lax.axis_index("c")
    sid = lax.axis_index("s")
    wid = sid * NC + cid
    s0 = pl.multiple_of(sid * NPS, NPS)

    pltpu.sync_copy(parts.at[0, pl.ds(s0, NPS)], pa)
    pltpu.sync_copy(parts.at[1, pl.ds(s0, NPS)], pb)
    pltpu.sync_copy(aux.at[pl.ds(s0, NPS)], yv)
    for k in range(NPS // L):
        sl = pl.ds(k * L, L)
        a = pa[sl]
        b_ = pb[sl]
        if first_hop:
            uv[sl] = a * b_
        else:
            c_ = yv[sl]
            uv[sl] = c_ * c_ * (a + b_)
    pltpu.sync_copy(uv, u_sh.at[pl.ds(s0, NPS)])

    pltpu.sync_copy(src3.at[wid], src_v)
    pltpu.sync_copy(dst3.at[wid], dst_v)
    plsc.subcore_barrier()

    pltpu.sync_copy(u_sh, u_full)

    @pl.when(jnp.logical_and(sid == 0, cid == 0))
    def _():
        pltpu.sync_copy(u_full, z_sh)

    @pl.when(jnp.logical_and(sid == 0, cid == 1))
    def _():
        pltpu.sync_copy(zeros1, zb)
        pltpu.sync_copy(zb, z_sh)

    plsc.subcore_barrier()

    def fill(j, carry):
        for c in range(128 // L):
            sl = pl.ds(c * L, L)
            vals_v[j, sl] = plsc.load_gather(u_full, [src_v[j, sl]])
        pltpu.async_copy(vals_v.at[j], z_sh.at[dst_v.at[j]], sem, add=True)
        return carry

    lax.fori_loop(0, ROWS, fill, 0)

    def drain(j, carry):
        pltpu.make_async_copy(vals_v.at[j], z_sh.at[dst_v.at[j]], sem).wait()
        return carry

    lax.fori_loop(0, ROWS, drain, 0)
    plsc.subcore_barrier()

    pltpu.sync_copy(z_sh.at[pl.ds(s0, NPS)], uv)
    pltpu.sync_copy(uv, tp_out.at[cid, pl.ds(s0, NPS)])


def _hop_scratch():
    return [
        pltpu.VMEM((ROWS, 128), jnp.int32),
        pltpu.VMEM((ROWS, 128), jnp.int32),
        pltpu.VMEM((ROWS, 128), jnp.float32),
        pltpu.VMEM((NPS,), jnp.float32),
        pltpu.VMEM((NPS,), jnp.float32),
        pltpu.VMEM((NPS,), jnp.float32),
        pltpu.VMEM((NPS,), jnp.float32),
        pltpu.VMEM((NPAD,), jnp.float32),
        pltpu.VMEM((NPAD,), jnp.float32),
        pltpu.VMEM_SHARED((NPAD,), jnp.float32),
        pltpu.VMEM_SHARED((NPAD,), jnp.float32),
        pltpu.SemaphoreType.DMA,
    ]


@functools.lru_cache(maxsize=None)
def _sc_kernels():
    params = pltpu.CompilerParams(needs_layout_passes=False)
    k_deg = functools.partial(
        pl.kernel,
        mesh=_mesh(),
        compiler_params=params,
        out_type=[jax.ShapeDtypeStruct((NC, NPAD), jnp.float32)],
        scratch_types=[
            pltpu.VMEM((ROWS, 128), jnp.int32),
            pltpu.VMEM((ROWS, 128), jnp.float32),
            pltpu.VMEM((NPAD,), jnp.float32),
            pltpu.VMEM((NPS,), jnp.float32),
            pltpu.VMEM_SHARED((NPAD,), jnp.float32),
            pltpu.SemaphoreType.DMA,
        ],
    )(_k_deg_body)
    k_hop1 = functools.partial(
        pl.kernel,
        mesh=_mesh(),
        compiler_params=params,
        out_type=[jax.ShapeDtypeStruct((NC, NPAD), jnp.float32)],
        scratch_types=_hop_scratch(),
    )(functools.partial(_hop_body, True))
    k_hop2 = functools.partial(
        pl.kernel,
        mesh=_mesh(),
        compiler_params=params,
        out_type=[jax.ShapeDtypeStruct((NC, NPAD), jnp.float32)],
        scratch_types=_hop_scratch(),
    )(functools.partial(_hop_body, False))
    return k_deg, k_hop1, k_hop2


def kernel(x, edge_index, W, b):
    x = x.astype(jnp.float32)
    src = edge_index[0].astype(jnp.int32)
    dst = edge_index[1].astype(jnp.int32)

    n_pad_e = EPAD - N_EDGES
    pad_src = jnp.full((n_pad_e,), DEAD0, dtype=jnp.int32)
    pad_dst = DEAD0 + (jnp.arange(n_pad_e, dtype=jnp.int32) % (NPAD - DEAD0))
    src3 = jnp.concatenate([src, pad_src]).reshape(NW, ROWS, 128)
    dst3 = jnp.concatenate([dst, pad_dst]).reshape(NW, ROWS, 128)

    xpad = jnp.pad(x, ((0, NPAD - N_NODES), (0, 0)))
    wrow = W.astype(jnp.float32).reshape(1, D_FEAT)
    ones2 = jnp.ones((ROWS, 128), jnp.float32)
    zeros1 = jnp.zeros((NPAD,), jnp.float32)

    _k_deg, _k_hop1, _k_hop2 = _sc_kernels()
    degp = _k_deg(dst3, ones2, zeros1)
    if isinstance(degp, (tuple, list)):
        (degp,) = degp

    dp = degp.reshape(NC, ROWS, 128)
    y, d2 = pl.pallas_call(
        _mv_body,
        out_shape=[jax.ShapeDtypeStruct((NPAD, 1), jnp.float32),
                   jax.ShapeDtypeStruct((ROWS, 128), jnp.float32)],
    )(xpad, wrow, dp[0], dp[1])
    y1 = y.reshape(NPAD)
    dis = d2.reshape(NPAD)

    dy2 = jnp.stack([dis, y1])
    t1p = _k_hop1(src3, dst3, dy2, y1, zeros1)
    if isinstance(t1p, (tuple, list)):
        (t1p,) = t1p
    t2p = _k_hop2(src3, dst3, t1p, dis, zeros1)
    if isinstance(t2p, (tuple, list)):
        (t2p,) = t2p

    a2 = t2p.reshape(NC, ROWS, 128)
    bias2 = jnp.broadcast_to(b.astype(jnp.float32).reshape(1, 1), (ROWS, 128))
    o2 = pl.pallas_call(
        _fin_body,
        out_shape=jax.ShapeDtypeStruct((ROWS, 128), jnp.float32),
    )(a2[0], a2[1], d2, bias2)
    return o2.reshape(NPAD, 1)[:N_NODES]


# speedup vs baseline: 120.0350x; 1.0806x over previous
import functools

import jax
import jax.numpy as jnp
from jax import lax
from jax.experimental import pallas as pl
from jax.experimental.pallas import tpu as pltpu
from jax.experimental.pallas import tpu_sc as plsc

N_NODES = 10000
N_EDGES = 320000
D_FEAT = 128
NC, NS, L = 2, 16, 16
NPAD = 10240
ROWS = 160
EPT = ROWS * 128
EPAD = NS * EPT
NPS = NPAD // NS
DEAD0 = 10200


def _mesh():
    return plsc.VectorSubcoreMesh(core_axis_name="c", subcore_axis_name="s")


def _rsqrt16(d):
    i = plsc.bitcast(d, jnp.int32)
    i = jnp.int32(0x5F3759DF) - lax.shift_right_logical(i, jnp.int32(1))
    y = plsc.bitcast(i, jnp.float32)
    for _ in range(3):
        y = y * (1.5 - 0.5 * d * y * y)
    return y


def _mv_body(x_ref, w_ref, y_ref):
    y_ref[...] = jnp.sum(x_ref[...] * w_ref[...], axis=1, keepdims=True)


def _sgc_body(src3, dst3, y1, b16, ones2, zeros1, out,
              src_v, dst_v, vals_v, dis_v, sl_v, u_full, zb, b_v,
              u_sh, t_sh, w_sh, sem):
    cid = lax.axis_index("c")
    sid = lax.axis_index("s")
    s0 = pl.multiple_of(sid * NPS, NPS)

    pltpu.sync_copy(src3.at[sid], src_v)
    pltpu.sync_copy(dst3.at[sid], dst_v)
    pltpu.sync_copy(ones2, vals_v)
    pltpu.sync_copy(b16, b_v)

    @pl.when(sid == 0)
    def _():
        pltpu.sync_copy(zeros1, zb)
        pltpu.sync_copy(zb, w_sh)

    plsc.subcore_barrier()

    def scat(j, carry):
        pltpu.async_copy(vals_v.at[j], w_sh.at[dst_v.at[j]], sem, add=True)
        return carry

    lax.fori_loop(0, ROWS, scat, 0)

    def drain(j, carry):
        pltpu.make_async_copy(vals_v.at[j], w_sh.at[dst_v.at[j]], sem).wait()
        return carry

    lax.fori_loop(0, ROWS, drain, 0)
    plsc.subcore_barrier()

    pltpu.sync_copy(w_sh.at[pl.ds(s0, NPS)], sl_v)
    pltpu.sync_copy(y1.at[pl.ds(s0, NPS)], dis_v)
    for k in range(NPS // L):
        sl = pl.ds(k * L, L)
        dis = _rsqrt16(sl_v[sl] + 1.0)
        yk = dis_v[sl]
        dis_v[sl] = dis
        sl_v[sl] = dis * yk
    pltpu.sync_copy(sl_v, u_sh.at[pl.ds(s0, NPS)])
    pltpu.sync_copy(sl_v, t_sh.at[pl.ds(s0, NPS)])
    plsc.subcore_barrier()

    pltpu.sync_copy(u_sh, u_full)

    def fill1(j, carry):
        for c in range(128 // L):
            sl = pl.ds(c * L, L)
            vals_v[j, sl] = plsc.load_gather(u_full, [src_v[j, sl]])
        pltpu.async_copy(vals_v.at[j], t_sh.at[dst_v.at[j]], sem, add=True)
        return carry

    lax.fori_loop(0, ROWS, fill1, 0)

    def drain1(j, carry):
        pltpu.make_async_copy(vals_v.at[j], t_sh.at[dst_v.at[j]], sem).wait()
        return carry

    lax.fori_loop(0, ROWS, drain1, 0)
    plsc.subcore_barrier()

    pltpu.sync_copy(t_sh.at[pl.ds(s0, NPS)], sl_v)
    for k in range(NPS // L):
        sl = pl.ds(k * L, L)
        dis = dis_v[sl]
        sl_v[sl] = dis * dis * sl_v[sl]
    pltpu.sync_copy(sl_v, u_sh.at[pl.ds(s0, NPS)])
    pltpu.sync_copy(sl_v, w_sh.at[pl.ds(s0, NPS)])
    plsc.subcore_barrier()

    pltpu.sync_copy(u_sh, u_full)

    def fill2(j, carry):
        for c in range(128 // L):
            sl = pl.ds(c * L, L)
            vals_v[j, sl] = plsc.load_gather(u_full, [src_v[j, sl]])
        pltpu.async_copy(vals_v.at[j], w_sh.at[dst_v.at[j]], sem, add=True)
        return carry

    lax.fori_loop(0, ROWS, fill2, 0)

    def drain2(j, carry):
        pltpu.make_async_copy(vals_v.at[j], w_sh.at[dst_v.at[j]], sem).wait()
        return carry

    lax.fori_loop(0, ROWS, drain2, 0)
    plsc.subcore_barrier()

    @pl.when(cid == 0)
    def _():
        pltpu.sync_copy(w_sh.at[pl.ds(s0, NPS)], sl_v)
        for k in range(NPS // L):
            sl = pl.ds(k * L, L)
            h = dis_v[sl] * sl_v[sl] + b_v[...]
            sl_v[sl] = h * h
        pltpu.sync_copy(sl_v, out.at[pl.ds(s0, NPS)])


@functools.lru_cache(maxsize=None)
def _sc_kernel():
    return functools.partial(
        pl.kernel,
        mesh=_mesh(),
        compiler_params=pltpu.CompilerParams(needs_layout_passes=False),
        out_type=[jax.ShapeDtypeStruct((NPAD,), jnp.float32)],
        scratch_types=[
            pltpu.VMEM((ROWS, 128), jnp.int32),
            pltpu.VMEM((ROWS, 128), jnp.int32),
            pltpu.VMEM((ROWS, 128), jnp.float32),
            pltpu.VMEM((NPS,), jnp.float32),
            pltpu.VMEM((NPS,), jnp.float32),
            pltpu.VMEM((NPAD,), jnp.float32),
            pltpu.VMEM((NPAD,), jnp.float32),
            pltpu.VMEM((L,), jnp.float32),
            pltpu.VMEM_SHARED((NPAD,), jnp.float32),
            pltpu.VMEM_SHARED((NPAD,), jnp.float32),
            pltpu.VMEM_SHARED((NPAD,), jnp.float32),
            pltpu.SemaphoreType.DMA,
        ],
    )(_sgc_body)


def kernel(x, edge_index, W, b):
    x = x.astype(jnp.float32)
    src = edge_index[0].astype(jnp.int32)
    dst = edge_index[1].astype(jnp.int32)

    n_pad_e = EPAD - N_EDGES
    pad_src = jnp.full((n_pad_e,), DEAD0, dtype=jnp.int32)
    pad_dst = DEAD0 + (jnp.arange(n_pad_e, dtype=jnp.int32) % (NPAD - DEAD0))
    src3 = jnp.concatenate([src, pad_src]).reshape(NS, ROWS, 128)
    dst3 = jnp.concatenate([dst, pad_dst]).reshape(NS, ROWS, 128)

    xpad = jnp.pad(x, ((0, NPAD - N_NODES), (0, 0)))
    wrow = W.astype(jnp.float32).reshape(1, D_FEAT)
    b16 = jnp.broadcast_to(b.astype(jnp.float32).reshape(1), (L,))
    ones2 = jnp.ones((ROWS, 128), jnp.float32)
    zeros1 = jnp.zeros((NPAD,), jnp.float32)

    y = pl.pallas_call(
        _mv_body,
        out_shape=jax.ShapeDtypeStruct((NPAD, 1), jnp.float32),
    )(xpad, wrow)
    y1 = y.reshape(NPAD)

    o = _sc_kernel()(src3, dst3, y1, b16, ones2, zeros1)
    if isinstance(o, (tuple, list)):
        (o,) = o
    return o.reshape(NPAD, 1)[:N_NODES]


# speedup vs baseline: 128.8007x; 1.0730x over previous
import functools

import jax
import jax.numpy as jnp
from jax import lax
from jax.experimental import pallas as pl
from jax.experimental.pallas import tpu as pltpu
from jax.experimental.pallas import tpu_sc as plsc

N_NODES = 10000
N_EDGES = 320000
D_FEAT = 128
NC, NS, L = 2, 16, 16
NPAD = 10240
NB = NPAD // 128
R_TOT = 2560
ROWS = R_TOT // NS
R_REAL = N_EDGES // 128
NPS = NPAD // NS


def _mesh():
    return plsc.VectorSubcoreMesh(core_axis_name="c", subcore_axis_name="s")


def _rsqrt16(d):
    i = plsc.bitcast(d, jnp.int32)
    i = jnp.int32(0x5F3759DF) - lax.shift_right_logical(i, jnp.int32(1))
    y = plsc.bitcast(i, jnp.float32)
    for _ in range(3):
        y = y * (1.5 - 0.5 * d * y * y)
    return y


def _mv_body(x3_ref, w_ref, y_ref):
    y_ref[...] = jnp.sum(x3_ref[...] * w_ref[...][None], axis=2)


def _sgc_body(ei3, y2, b16, ones2, zeros1, out,
              src_v, dst_v, vals_v, dis_v, sl_v, yf_v, u_full, zb, b_v,
              u_sh, t_sh, w_sh, sem):
    cid = lax.axis_index("c")
    sid = lax.axis_index("s")
    s0 = pl.multiple_of(sid * NPS, NPS)
    rs = pl.multiple_of(sid * ROWS, ROWS)
    nrows = jnp.where(sid == NS - 1, R_REAL - (NS - 1) * ROWS, ROWS)

    with jax.named_scope("stage"):
        pltpu.sync_copy(ei3.at[0, pl.ds(rs, ROWS)], src_v)
        pltpu.sync_copy(ei3.at[1, pl.ds(rs, ROWS)], dst_v)
        pltpu.sync_copy(ones2, vals_v)
        pltpu.sync_copy(b16, b_v)

        @pl.when(sid == 0)
        def _():
            pltpu.sync_copy(zeros1, zb)
            pltpu.sync_copy(zb, w_sh)

    plsc.subcore_barrier()

    with jax.named_scope("deg_scatter"):
        def scat(j, carry):
            pltpu.async_copy(vals_v.at[j], w_sh.at[dst_v.at[j]], sem,
                             add=True)
            return carry

        lax.fori_loop(0, nrows, scat, 0)

        def drain(j, carry):
            pltpu.make_async_copy(vals_v.at[j], w_sh.at[dst_v.at[j]],
                                  sem).wait()
            return carry

        lax.fori_loop(0, nrows, drain, 0)
    plsc.subcore_barrier()

    with jax.named_scope("build_u0"):
        pltpu.sync_copy(w_sh.at[pl.ds(s0, NPS)], sl_v)
        pltpu.sync_copy(y2, yf_v)
        r0 = sid * (NPS // 128)
        for k in range(NPS // L):
            sl = pl.ds(k * L, L)
            dis = _rsqrt16(sl_v[sl] + 1.0)
            yk = yf_v[r0 + k // 8, pl.ds((k % 8) * L, L)]
            dis_v[sl] = dis
            sl_v[sl] = dis * yk
        pltpu.sync_copy(sl_v, u_sh.at[pl.ds(s0, NPS)])
        pltpu.sync_copy(sl_v, t_sh.at[pl.ds(s0, NPS)])
    plsc.subcore_barrier()

    with jax.named_scope("hop1"):
        pltpu.sync_copy(u_sh, u_full)

        def fill1(j, carry):
            for c in range(128 // L):
                sl = pl.ds(c * L, L)
                vals_v[j, sl] = plsc.load_gather(u_full, [src_v[j, sl]])
            pltpu.async_copy(vals_v.at[j], t_sh.at[dst_v.at[j]], sem,
                             add=True)
            return carry

        lax.fori_loop(0, nrows, fill1, 0)

        def drain1(j, carry):
            pltpu.make_async_copy(vals_v.at[j], t_sh.at[dst_v.at[j]],
                                  sem).wait()
            return carry

        lax.fori_loop(0, nrows, drain1, 0)
    plsc.subcore_barrier()

    with jax.named_scope("build_u1"):
        pltpu.sync_copy(t_sh.at[pl.ds(s0, NPS)], sl_v)
        for k in range(NPS // L):
            sl = pl.ds(k * L, L)
            dis = dis_v[sl]
            sl_v[sl] = dis * dis * sl_v[sl]
        pltpu.sync_copy(sl_v, u_sh.at[pl.ds(s0, NPS)])
        pltpu.sync_copy(sl_v, w_sh.at[pl.ds(s0, NPS)])
    plsc.subcore_barrier()

    with jax.named_scope("hop2"):
        pltpu.sync_copy(u_sh, u_full)

        def fill2(j, carry):
            for c in range(128 // L):
                sl = pl.ds(c * L, L)
                vals_v[j, sl] = plsc.load_gather(u_full, [src_v[j, sl]])
            pltpu.async_copy(vals_v.at[j], w_sh.at[dst_v.at[j]], sem,
                             add=True)
            return carry

        lax.fori_loop(0, nrows, fill2, 0)

        def drain2(j, carry):
            pltpu.make_async_copy(vals_v.at[j], w_sh.at[dst_v.at[j]],
                                  sem).wait()
            return carry

        lax.fori_loop(0, nrows, drain2, 0)
    plsc.subcore_barrier()

    with jax.named_scope("epilogue"):
        @pl.when(cid == 0)
        def _():
            pltpu.sync_copy(w_sh.at[pl.ds(s0, NPS)], sl_v)
            for k in range(NPS // L):
                sl = pl.ds(k * L, L)
                h = dis_v[sl] * sl_v[sl] + b_v[...]
                sl_v[sl] = h * h
            pltpu.sync_copy(sl_v, out.at[pl.ds(s0, NPS)])


@functools.lru_cache(maxsize=None)
def _sc_kernel():
    return functools.partial(
        pl.kernel,
        mesh=_mesh(),
        compiler_params=pltpu.CompilerParams(needs_layout_passes=False),
        out_type=[jax.ShapeDtypeStruct((NPAD,), jnp.float32)],
        scratch_types=[
            pltpu.VMEM((ROWS, 128), jnp.int32),
            pltpu.VMEM((ROWS, 128), jnp.int32),
            pltpu.VMEM((ROWS, 128), jnp.float32),
            pltpu.VMEM((NPS,), jnp.float32),
            pltpu.VMEM((NPS,), jnp.float32),
            pltpu.VMEM((NB, 128), jnp.float32),
            pltpu.VMEM((NPAD,), jnp.float32),
            pltpu.VMEM((NPAD,), jnp.float32),
            pltpu.VMEM((L,), jnp.float32),
            pltpu.VMEM_SHARED((NPAD,), jnp.float32),
            pltpu.VMEM_SHARED((NPAD,), jnp.float32),
            pltpu.VMEM_SHARED((NPAD,), jnp.float32),
            pltpu.SemaphoreType.DMA,
        ],
    )(_sgc_body)


def kernel(x, edge_index, W, b):
    ei3 = jnp.pad(edge_index.astype(jnp.int32),
                  ((0, 0), (0, R_TOT * 128 - N_EDGES))).reshape(2, R_TOT, 128)
    xpad = jnp.pad(x.astype(jnp.float32), ((0, NPAD - N_NODES), (0, 0)))
    x3 = xpad.reshape(NB, 128, D_FEAT)
    wrow = W.astype(jnp.float32).reshape(1, D_FEAT)
    b16 = jnp.broadcast_to(b.astype(jnp.float32).reshape(1), (L,))
    ones2 = jnp.ones((ROWS, 128), jnp.float32)
    zeros1 = jnp.zeros((NPAD,), jnp.float32)

    y2 = pl.pallas_call(
        _mv_body,
        out_shape=jax.ShapeDtypeStruct((NB, 128), jnp.float32),
    )(x3, wrow)

    o = _sc_kernel()(ei3, y2, b16, ones2, zeros1)
    if isinstance(o, (tuple, list)):
        (o,) = o
    return o.reshape(NPAD, 1)[:N_NODES]


# speedup vs baseline: 140.4800x; 1.0907x over previous
import functools

import jax
import jax.numpy as jnp
from jax import lax
from jax.experimental import pallas as pl
from jax.experimental.pallas import tpu as pltpu
from jax.experimental.pallas import tpu_sc as plsc

N_NODES = 10000
N_EDGES = 320000
D_FEAT = 128
NC, NS, L = 2, 16, 16
NPAD = 10240
NB = NPAD // 128
R_TOT = 2560
ROWS = R_TOT // NS
R_REAL = N_EDGES // 128
NPS = NPAD // NS


def _mesh():
    return plsc.VectorSubcoreMesh(core_axis_name="c", subcore_axis_name="s")


def _rsqrt16(d):
    i = plsc.bitcast(d, jnp.int32)
    i = jnp.int32(0x5F3759DF) - lax.shift_right_logical(i, jnp.int32(1))
    y = plsc.bitcast(i, jnp.float32)
    for _ in range(3):
        y = y * (1.5 - 0.5 * d * y * y)
    return y


def _mv_body(x3_ref, w_ref, y_ref):
    y_ref[...] = jnp.sum(x3_ref[...] * w_ref[...][None], axis=2)


def _sgc_body(ei3, y2, b16, zeros1, out,
              src_v, dst_v, vals_v, dis_v, sl_v, yf_v, u_full, zb, b_v,
              ones_v, u_sh, t_sh, w_sh, sem, sem_s, sem_d, sem_y):
    cid = lax.axis_index("c")
    sid = lax.axis_index("s")
    s0 = pl.multiple_of(sid * NPS, NPS)
    rs = pl.multiple_of(sid * ROWS, ROWS)
    nrows = jnp.where(sid == NS - 1, R_REAL - (NS - 1) * ROWS, ROWS)

    with jax.named_scope("stage"):
        pltpu.async_copy(ei3.at[0, pl.ds(rs, ROWS)], src_v, sem_s)
        pltpu.async_copy(ei3.at[1, pl.ds(rs, ROWS)], dst_v, sem_d)
        pltpu.async_copy(y2, yf_v, sem_y)
        pltpu.sync_copy(b16, b_v)
        for c in range(128 // L):
            ones_v[pl.ds(c * L, L)] = jnp.full((L,), 1.0, jnp.float32)

        @pl.when(sid == 0)
        def _():
            pltpu.sync_copy(zeros1, zb)
            pltpu.sync_copy(zb, w_sh)

    plsc.subcore_barrier()

    with jax.named_scope("deg_scatter"):
        pltpu.make_async_copy(ei3.at[1, pl.ds(rs, ROWS)], dst_v, sem_d).wait()

        def scat(j, carry):
            pltpu.async_copy(ones_v, w_sh.at[dst_v.at[j]], sem,
                             add=True)
            return carry

        lax.fori_loop(0, nrows, scat, 0)

        def drain(j, carry):
            pltpu.make_async_copy(ones_v, w_sh.at[dst_v.at[j]],
                                  sem).wait()
            return carry

        lax.fori_loop(0, nrows, drain, 0)
    plsc.subcore_barrier()

    with jax.named_scope("build_u0"):
        pltpu.sync_copy(w_sh.at[pl.ds(s0, NPS)], sl_v)
        pltpu.make_async_copy(y2, yf_v, sem_y).wait()
        r0 = sid * (NPS // 128)
        for k in range(NPS // L):
            sl = pl.ds(k * L, L)
            dis = _rsqrt16(sl_v[sl] + 1.0)
            yk = yf_v[r0 + k // 8, pl.ds((k % 8) * L, L)]
            dis_v[sl] = dis
            sl_v[sl] = dis * yk
        pltpu.sync_copy(sl_v, u_sh.at[pl.ds(s0, NPS)])
        pltpu.sync_copy(sl_v, t_sh.at[pl.ds(s0, NPS)])
    plsc.subcore_barrier()

    with jax.named_scope("hop1"):
        pltpu.make_async_copy(ei3.at[0, pl.ds(rs, ROWS)], src_v, sem_s).wait()
        pltpu.sync_copy(u_sh, u_full)

        def fill1(j, carry):
            for c in range(128 // L):
                sl = pl.ds(c * L, L)
                vals_v[j, sl] = plsc.load_gather(u_full, [src_v[j, sl]])
            pltpu.async_copy(vals_v.at[j], t_sh.at[dst_v.at[j]], sem,
                             add=True)
            return carry

        lax.fori_loop(0, nrows, fill1, 0)

        def drain1(j, carry):
            pltpu.make_async_copy(vals_v.at[j], t_sh.at[dst_v.at[j]],
                                  sem).wait()
            return carry

        lax.fori_loop(0, nrows, drain1, 0)
    plsc.subcore_barrier()

    with jax.named_scope("build_u1"):
        pltpu.sync_copy(t_sh.at[pl.ds(s0, NPS)], sl_v)
        for k in range(NPS // L):
            sl = pl.ds(k * L, L)
            dis = dis_v[sl]
            sl_v[sl] = dis * dis * sl_v[sl]
        pltpu.sync_copy(sl_v, u_sh.at[pl.ds(s0, NPS)])
        pltpu.sync_copy(sl_v, w_sh.at[pl.ds(s0, NPS)])
    plsc.subcore_barrier()

    with jax.named_scope("hop2"):
        pltpu.sync_copy(u_sh, u_full)

        def fill2(j, carry):
            for c in range(128 // L):
                sl = pl.ds(c * L, L)
                vals_v[j, sl] = plsc.load_gather(u_full, [src_v[j, sl]])
            pltpu.async_copy(vals_v.at[j], w_sh.at[dst_v.at[j]], sem,
                             add=True)
            return carry

        lax.fori_loop(0, nrows, fill2, 0)

        def drain2(j, carry):
            pltpu.make_async_copy(vals_v.at[j], w_sh.at[dst_v.at[j]],
                                  sem).wait()
            return carry

        lax.fori_loop(0, nrows, drain2, 0)
    plsc.subcore_barrier()

    with jax.named_scope("epilogue"):
        @pl.when(cid == 0)
        def _():
            pltpu.sync_copy(w_sh.at[pl.ds(s0, NPS)], sl_v)
            for k in range(NPS // L):
                sl = pl.ds(k * L, L)
                h = dis_v[sl] * sl_v[sl] + b_v[...]
                sl_v[sl] = h * h
            pltpu.sync_copy(sl_v, out.at[pl.ds(s0, NPS)])


@functools.lru_cache(maxsize=None)
def _sc_kernel():
    return functools.partial(
        pl.kernel,
        mesh=_mesh(),
        compiler_params=pltpu.CompilerParams(needs_layout_passes=False),
        out_type=[jax.ShapeDtypeStruct((NPAD,), jnp.float32)],
        scratch_types=[
            pltpu.VMEM((ROWS, 128), jnp.int32),
            pltpu.VMEM((ROWS, 128), jnp.int32),
            pltpu.VMEM((ROWS, 128), jnp.float32),
            pltpu.VMEM((NPS,), jnp.float32),
            pltpu.VMEM((NPS,), jnp.float32),
            pltpu.VMEM((NB, 128), jnp.float32),
            pltpu.VMEM((NPAD,), jnp.float32),
            pltpu.VMEM((NPAD,), jnp.float32),
            pltpu.VMEM((L,), jnp.float32),
            pltpu.VMEM((128,), jnp.float32),
            pltpu.VMEM_SHARED((NPAD,), jnp.float32),
            pltpu.VMEM_SHARED((NPAD,), jnp.float32),
            pltpu.VMEM_SHARED((NPAD,), jnp.float32),
            pltpu.SemaphoreType.DMA,
            pltpu.SemaphoreType.DMA,
            pltpu.SemaphoreType.DMA,
            pltpu.SemaphoreType.DMA,
        ],
    )(_sgc_body)


def kernel(x, edge_index, W, b):
    ei3 = jnp.pad(edge_index.astype(jnp.int32),
                  ((0, 0), (0, R_TOT * 128 - N_EDGES))).reshape(2, R_TOT, 128)
    xpad = jnp.pad(x.astype(jnp.float32), ((0, NPAD - N_NODES), (0, 0)))
    x3 = xpad.reshape(NB, 128, D_FEAT)
    wrow = W.astype(jnp.float32).reshape(1, D_FEAT)
    b16 = jnp.broadcast_to(b.astype(jnp.float32).reshape(1), (L,))
    zeros1 = jnp.zeros((NPAD,), jnp.float32)

    y2 = pl.pallas_call(
        _mv_body,
        out_shape=jax.ShapeDtypeStruct((NB, 128), jnp.float32),
    )(x3, wrow)

    o = _sc_kernel()(ei3, y2, b16, zeros1)
    if isinstance(o, (tuple, list)):
        (o,) = o
    return o.reshape(NPAD, 1)[:N_NODES]


# speedup vs baseline: 166.8853x; 1.1880x over previous
import functools

import jax
import jax.numpy as jnp
from jax import lax
from jax.experimental import pallas as pl
from jax.experimental.pallas import tpu as pltpu
from jax.experimental.pallas import tpu_sc as plsc

N_NODES = 10000
N_EDGES = 320000
D_FEAT = 128
NC, NS, L = 2, 16, 16
NPAD = 10240
NB = NPAD // 128
R_TOT = 2560
ROWS = R_TOT // NS
R_REAL = N_EDGES // 128
NPS = NPAD // NS


def _mesh():
    return plsc.VectorSubcoreMesh(core_axis_name="c", subcore_axis_name="s")


def _rsqrt16(d):
    i = plsc.bitcast(d, jnp.int32)
    i = jnp.int32(0x5F3759DF) - lax.shift_right_logical(i, jnp.int32(1))
    y = plsc.bitcast(i, jnp.float32)
    for _ in range(3):
        y = y * (1.5 - 0.5 * d * y * y)
    return y


def _mv_body(x3_ref, w_ref, y_ref):
    y_ref[...] = jnp.sum(x3_ref[...] * w_ref[...][None], axis=2)


def _sgc_body(ei3, y2, b16, zeros1, out,
              src_v, dst_v, vals_v, dis_v, sl_v, yf_v, u_full, zb, b_v,
              ones_v, u_sh, t_sh, w_sh, sem, sem_s, sem_d, sem_y):
    cid = lax.axis_index("c")
    sid = lax.axis_index("s")
    s0 = pl.multiple_of(sid * NPS, NPS)
    rs = pl.multiple_of(sid * ROWS, ROWS)
    nrows = jnp.where(sid == NS - 1, R_REAL - (NS - 1) * ROWS, ROWS)

    with jax.named_scope("stage"):
        pltpu.async_copy(ei3.at[0, pl.ds(rs, ROWS)], src_v, sem_s)
        pltpu.async_copy(ei3.at[1, pl.ds(rs, ROWS)], dst_v, sem_d)
        pltpu.async_copy(y2, yf_v, sem_y)
        pltpu.sync_copy(b16, b_v)
        for c in range(128 // L):
            ones_v[pl.ds(c * L, L)] = jnp.full((L,), 1.0, jnp.float32)

        @pl.when(sid == 0)
        def _():
            pltpu.sync_copy(zeros1, zb)
            pltpu.sync_copy(zb, w_sh)

    plsc.subcore_barrier()

    with jax.named_scope("deg_scatter"):
        pltpu.make_async_copy(ei3.at[1, pl.ds(rs, ROWS)], dst_v, sem_d).wait()

        @plsc.parallel_loop(0, nrows, unroll=4)
        def _(j):
            pltpu.async_copy(ones_v, w_sh.at[dst_v.at[j]], sem,
                             add=True)

        def drain(j, carry):
            pltpu.make_async_copy(ones_v, w_sh.at[dst_v.at[j]],
                                  sem).wait()
            return carry

        lax.fori_loop(0, nrows, drain, 0)
    plsc.subcore_barrier()

    with jax.named_scope("build_u0"):
        pltpu.sync_copy(w_sh.at[pl.ds(s0, NPS)], sl_v)
        pltpu.make_async_copy(y2, yf_v, sem_y).wait()
        r0 = sid * (NPS // 128)
        for k in range(NPS // L):
            sl = pl.ds(k * L, L)
            dis = _rsqrt16(sl_v[sl] + 1.0)
            yk = yf_v[r0 + k // 8, pl.ds((k % 8) * L, L)]
            dis_v[sl] = dis
            sl_v[sl] = dis * yk
        pltpu.sync_copy(sl_v, u_sh.at[pl.ds(s0, NPS)])
        pltpu.sync_copy(sl_v, t_sh.at[pl.ds(s0, NPS)])
    plsc.subcore_barrier()

    with jax.named_scope("hop1"):
        pltpu.make_async_copy(ei3.at[0, pl.ds(rs, ROWS)], src_v, sem_s).wait()
        pltpu.sync_copy(u_sh, u_full)

        @plsc.parallel_loop(0, nrows, unroll=2)
        def _(j):
            for c in range(128 // L):
                sl = pl.ds(c * L, L)
                vals_v[j, sl] = plsc.load_gather(u_full, [src_v[j, sl]])
            pltpu.async_copy(vals_v.at[j], t_sh.at[dst_v.at[j]], sem,
                             add=True)

        def drain1(j, carry):
            pltpu.make_async_copy(vals_v.at[j], t_sh.at[dst_v.at[j]],
                                  sem).wait()
            return carry

        lax.fori_loop(0, nrows, drain1, 0)
    plsc.subcore_barrier()

    with jax.named_scope("build_u1"):
        pltpu.sync_copy(t_sh.at[pl.ds(s0, NPS)], sl_v)
        for k in range(NPS // L):
            sl = pl.ds(k * L, L)
            dis = dis_v[sl]
            sl_v[sl] = dis * dis * sl_v[sl]
        pltpu.sync_copy(sl_v, u_sh.at[pl.ds(s0, NPS)])
        pltpu.sync_copy(sl_v, w_sh.at[pl.ds(s0, NPS)])
    plsc.subcore_barrier()

    with jax.named_scope("hop2"):
        pltpu.sync_copy(u_sh, u_full)

        @plsc.parallel_loop(0, nrows, unroll=2)
        def _(j):
            for c in range(128 // L):
                sl = pl.ds(c * L, L)
                vals_v[j, sl] = plsc.load_gather(u_full, [src_v[j, sl]])
            pltpu.async_copy(vals_v.at[j], w_sh.at[dst_v.at[j]], sem,
                             add=True)

        def drain2(j, carry):
            pltpu.make_async_copy(vals_v.at[j], w_sh.at[dst_v.at[j]],
                                  sem).wait()
            return carry

        lax.fori_loop(0, nrows, drain2, 0)
    plsc.subcore_barrier()

    with jax.named_scope("epilogue"):
        @pl.when(cid == 0)
        def _():
            pltpu.sync_copy(w_sh.at[pl.ds(s0, NPS)], sl_v)
            for k in range(NPS // L):
                sl = pl.ds(k * L, L)
                h = dis_v[sl] * sl_v[sl] + b_v[...]
                sl_v[sl] = h * h
            pltpu.sync_copy(sl_v, out.at[pl.ds(s0, NPS)])


@functools.lru_cache(maxsize=None)
def _sc_kernel():
    return functools.partial(
        pl.kernel,
        mesh=_mesh(),
        compiler_params=pltpu.CompilerParams(needs_layout_passes=False),
        out_type=[jax.ShapeDtypeStruct((NPAD,), jnp.float32)],
        scratch_types=[
            pltpu.VMEM((ROWS, 128), jnp.int32),
            pltpu.VMEM((ROWS, 128), jnp.int32),
            pltpu.VMEM((ROWS, 128), jnp.float32),
            pltpu.VMEM((NPS,), jnp.float32),
            pltpu.VMEM((NPS,), jnp.float32),
            pltpu.VMEM((NB, 128), jnp.float32),
            pltpu.VMEM((NPAD,), jnp.float32),
            pltpu.VMEM((NPAD,), jnp.float32),
            pltpu.VMEM((L,), jnp.float32),
            pltpu.VMEM((128,), jnp.float32),
            pltpu.VMEM_SHARED((NPAD,), jnp.float32),
            pltpu.VMEM_SHARED((NPAD,), jnp.float32),
            pltpu.VMEM_SHARED((NPAD,), jnp.float32),
            pltpu.SemaphoreType.DMA,
            pltpu.SemaphoreType.DMA,
            pltpu.SemaphoreType.DMA,
            pltpu.SemaphoreType.DMA,
        ],
    )(_sgc_body)


def kernel(x, edge_index, W, b):
    ei3 = jnp.pad(edge_index.astype(jnp.int32),
                  ((0, 0), (0, R_TOT * 128 - N_EDGES))).reshape(2, R_TOT, 128)
    xpad = jnp.pad(x.astype(jnp.float32), ((0, NPAD - N_NODES), (0, 0)))
    x3 = xpad.reshape(NB, 128, D_FEAT)
    wrow = W.astype(jnp.float32).reshape(1, D_FEAT)
    b16 = jnp.broadcast_to(b.astype(jnp.float32).reshape(1), (L,))
    zeros1 = jnp.zeros((NPAD,), jnp.float32)

    y2 = pl.pallas_call(
        _mv_body,
        out_shape=jax.ShapeDtypeStruct((NB, 128), jnp.float32),
    )(x3, wrow)

    o = _sc_kernel()(ei3, y2, b16, zeros1)
    if isinstance(o, (tuple, list)):
        (o,) = o
    return o.reshape(NPAD, 1)[:N_NODES]


# speedup vs baseline: 167.0901x; 1.0012x over previous
import functools

import jax
import jax.numpy as jnp
from jax import lax
from jax.experimental import pallas as pl
from jax.experimental.pallas import tpu as pltpu
from jax.experimental.pallas import tpu_sc as plsc

N_NODES = 10000
N_EDGES = 320000
D_FEAT = 128
NC, NS, L = 2, 16, 16
NPAD = 10240
NB = NPAD // 128
R_TOT = 2560
ROWS = R_TOT // NS
R_REAL = N_EDGES // 128
NPS = NPAD // NS


def _mesh():
    return plsc.VectorSubcoreMesh(core_axis_name="c", subcore_axis_name="s")


def _rsqrt16(d):
    i = plsc.bitcast(d, jnp.int32)
    i = jnp.int32(0x5F3759DF) - lax.shift_right_logical(i, jnp.int32(1))
    y = plsc.bitcast(i, jnp.float32)
    for _ in range(3):
        y = y * (1.5 - 0.5 * d * y * y)
    return y


def _mv_body(x3_ref, w_ref, y_ref):
    y_ref[...] = jnp.sum(x3_ref[...] * w_ref[...][None], axis=2)


def _sgc_body(ei3, y2, b16, zeros1, out,
              src_v, dst_v, vals_v, dis_v, sl_v, yf_v, u_full, zb, b_v,
              ones_v, u_sh, t_sh, w_sh, sem, sem_s, sem_d, sem_y):
    cid = lax.axis_index("c")
    sid = lax.axis_index("s")
    s0 = pl.multiple_of(sid * NPS, NPS)
    rs = pl.multiple_of(sid * ROWS, ROWS)
    nrows = jnp.where(sid == NS - 1, R_REAL - (NS - 1) * ROWS, ROWS)

    with jax.named_scope("stage"):
        pltpu.async_copy(ei3.at[0, pl.ds(rs, ROWS)], src_v, sem_s)
        pltpu.async_copy(ei3.at[1, pl.ds(rs, ROWS)], dst_v, sem_d)
        pltpu.async_copy(y2, yf_v, sem_y)
        pltpu.sync_copy(b16, b_v)
        for c in range(128 // L):
            ones_v[pl.ds(c * L, L)] = jnp.full((L,), 1.0, jnp.float32)

        @pl.when(sid == 0)
        def _():
            pltpu.sync_copy(zeros1, zb)
            pltpu.sync_copy(zb, w_sh)

    plsc.subcore_barrier()

    with jax.named_scope("deg_scatter"):
        pltpu.make_async_copy(ei3.at[1, pl.ds(rs, ROWS)], dst_v, sem_d).wait()

        @plsc.parallel_loop(0, nrows, unroll=8)
        def _(j):
            pltpu.async_copy(ones_v, w_sh.at[dst_v.at[j]], sem,
                             add=True)

        def drain(j, carry):
            pltpu.make_async_copy(ones_v, w_sh.at[dst_v.at[j]],
                                  sem).wait()
            return carry

        lax.fori_loop(0, nrows, drain, 0)
    plsc.subcore_barrier()

    with jax.named_scope("build_u0"):
        pltpu.sync_copy(w_sh.at[pl.ds(s0, NPS)], sl_v)
        pltpu.make_async_copy(y2, yf_v, sem_y).wait()
        r0 = sid * (NPS // 128)
        for k in range(NPS // L):
            sl = pl.ds(k * L, L)
            dis = _rsqrt16(sl_v[sl] + 1.0)
            yk = yf_v[r0 + k // 8, pl.ds((k % 8) * L, L)]
            dis_v[sl] = dis
            sl_v[sl] = dis * yk
        pltpu.sync_copy(sl_v, u_sh.at[pl.ds(s0, NPS)])
        pltpu.sync_copy(sl_v, t_sh.at[pl.ds(s0, NPS)])
    plsc.subcore_barrier()

    with jax.named_scope("hop1"):
        pltpu.make_async_copy(ei3.at[0, pl.ds(rs, ROWS)], src_v, sem_s).wait()
        pltpu.sync_copy(u_sh, u_full)

        @plsc.parallel_loop(0, nrows, unroll=4)
        def _(j):
            for c in range(128 // L):
                sl = pl.ds(c * L, L)
                vals_v[j, sl] = plsc.load_gather(u_full, [src_v[j, sl]])
            pltpu.async_copy(vals_v.at[j], t_sh.at[dst_v.at[j]], sem,
                             add=True)

        def drain1(j, carry):
            pltpu.make_async_copy(vals_v.at[j], t_sh.at[dst_v.at[j]],
                                  sem).wait()
            return carry

        lax.fori_loop(0, nrows, drain1, 0)
    plsc.subcore_barrier()

    with jax.named_scope("build_u1"):
        pltpu.sync_copy(t_sh.at[pl.ds(s0, NPS)], sl_v)
        for k in range(NPS // L):
            sl = pl.ds(k * L, L)
            dis = dis_v[sl]
            sl_v[sl] = dis * dis * sl_v[sl]
        pltpu.sync_copy(sl_v, u_sh.at[pl.ds(s0, NPS)])
        pltpu.sync_copy(sl_v, w_sh.at[pl.ds(s0, NPS)])
    plsc.subcore_barrier()

    with jax.named_scope("hop2"):
        pltpu.sync_copy(u_sh, u_full)

        @plsc.parallel_loop(0, nrows, unroll=4)
        def _(j):
            for c in range(128 // L):
                sl = pl.ds(c * L, L)
                vals_v[j, sl] = plsc.load_gather(u_full, [src_v[j, sl]])
            pltpu.async_copy(vals_v.at[j], w_sh.at[dst_v.at[j]], sem,
                             add=True)

        def drain2(j, carry):
            pltpu.make_async_copy(vals_v.at[j], w_sh.at[dst_v.at[j]],
                                  sem).wait()
            return carry

        lax.fori_loop(0, nrows, drain2, 0)
    plsc.subcore_barrier()

    with jax.named_scope("epilogue"):
        @pl.when(cid == 0)
        def _():
            pltpu.sync_copy(w_sh.at[pl.ds(s0, NPS)], sl_v)
            for k in range(NPS // L):
                sl = pl.ds(k * L, L)
                h = dis_v[sl] * sl_v[sl] + b_v[...]
                sl_v[sl] = h * h
            pltpu.sync_copy(sl_v, out.at[pl.ds(s0, NPS)])


@functools.lru_cache(maxsize=None)
def _sc_kernel():
    return functools.partial(
        pl.kernel,
        mesh=_mesh(),
        compiler_params=pltpu.CompilerParams(needs_layout_passes=False),
        out_type=[jax.ShapeDtypeStruct((NPAD,), jnp.float32)],
        scratch_types=[
            pltpu.VMEM((ROWS, 128), jnp.int32),
            pltpu.VMEM((ROWS, 128), jnp.int32),
            pltpu.VMEM((ROWS, 128), jnp.float32),
            pltpu.VMEM((NPS,), jnp.float32),
            pltpu.VMEM((NPS,), jnp.float32),
            pltpu.VMEM((NB, 128), jnp.float32),
            pltpu.VMEM((NPAD,), jnp.float32),
            pltpu.VMEM((NPAD,), jnp.float32),
            pltpu.VMEM((L,), jnp.float32),
            pltpu.VMEM((128,), jnp.float32),
            pltpu.VMEM_SHARED((NPAD,), jnp.float32),
            pltpu.VMEM_SHARED((NPAD,), jnp.float32),
            pltpu.VMEM_SHARED((NPAD,), jnp.float32),
            pltpu.SemaphoreType.DMA,
            pltpu.SemaphoreType.DMA,
            pltpu.SemaphoreType.DMA,
            pltpu.SemaphoreType.DMA,
        ],
    )(_sgc_body)


def kernel(x, edge_index, W, b):
    ei3 = jnp.pad(edge_index.astype(jnp.int32),
                  ((0, 0), (0, R_TOT * 128 - N_EDGES))).reshape(2, R_TOT, 128)
    xpad = jnp.pad(x.astype(jnp.float32), ((0, NPAD - N_NODES), (0, 0)))
    x3 = xpad.reshape(NB, 128, D_FEAT)
    wrow = W.astype(jnp.float32).reshape(1, D_FEAT)
    b16 = jnp.broadcast_to(b.astype(jnp.float32).reshape(1), (L,))
    zeros1 = jnp.zeros((NPAD,), jnp.float32)

    y2 = pl.pallas_call(
        _mv_body,
        out_shape=jax.ShapeDtypeStruct((NB, 128), jnp.float32),
    )(x3, wrow)

    o = _sc_kernel()(ei3, y2, b16, zeros1)
    if isinstance(o, (tuple, list)):
        (o,) = o
    return o.reshape(NPAD, 1)[:N_NODES]


# speedup vs baseline: 181.6375x; 1.0871x over previous
import functools

import jax
import jax.numpy as jnp
from jax import lax
from jax.experimental import pallas as pl
from jax.experimental.pallas import tpu as pltpu
from jax.experimental.pallas import tpu_sc as plsc

N_NODES = 10000
N_EDGES = 320000
D_FEAT = 128
NC, NS, L = 2, 16, 16
NPAD = 10240
NB = NPAD // 128
R_TOT = 2560
ROWS = R_TOT // NS
R_REAL = N_EDGES // 128
NPS = NPAD // NS


def _mesh():
    return plsc.VectorSubcoreMesh(core_axis_name="c", subcore_axis_name="s")


def _rsqrt16(d):
    i = plsc.bitcast(d, jnp.int32)
    i = jnp.int32(0x5F3759DF) - lax.shift_right_logical(i, jnp.int32(1))
    y = plsc.bitcast(i, jnp.float32)
    for _ in range(3):
        y = y * (1.5 - 0.5 * d * y * y)
    return y


def _mv_body(x3_ref, w_ref, y_ref):
    y_ref[...] = jnp.sum(x3_ref[...] * w_ref[...][None], axis=2)


def _deg_body(ei3, zeros1, degp, dst_v, zb, sl_v, ones_v, w_sh, sem, sem_d):
    cid = lax.axis_index("c")
    sid = lax.axis_index("s")
    s0 = pl.multiple_of(sid * NPS, NPS)
    hrows = ROWS // 2
    rs = pl.multiple_of(cid * (R_TOT // 2) + sid * hrows, hrows)
    nrows = jnp.where(jnp.logical_and(cid == 1, sid == NS - 1),
                      R_REAL - (R_TOT // 2) - (NS - 1) * hrows, hrows)

    pltpu.async_copy(ei3.at[1, pl.ds(rs, hrows)], dst_v, sem_d)
    for c in range(128 // L):
        ones_v[pl.ds(c * L, L)] = jnp.full((L,), 1.0, jnp.float32)

    @pl.when(sid == 0)
    def _():
        pltpu.sync_copy(zeros1, zb)
        pltpu.sync_copy(zb, w_sh)

    plsc.subcore_barrier()
    pltpu.make_async_copy(ei3.at[1, pl.ds(rs, hrows)], dst_v, sem_d).wait()

    @plsc.parallel_loop(0, nrows, unroll=8)
    def _(j):
        pltpu.async_copy(ones_v, w_sh.at[dst_v.at[j]], sem, add=True)

    def drain(j, carry):
        pltpu.make_async_copy(ones_v, w_sh.at[dst_v.at[j]], sem).wait()
        return carry

    lax.fori_loop(0, nrows, drain, 0)
    plsc.subcore_barrier()
    pltpu.sync_copy(w_sh.at[pl.ds(s0, NPS)], sl_v)
    pltpu.sync_copy(sl_v, degp.at[cid, pl.ds(s0, NPS)])


def _sgc_body(ei3, y2, degp, b16, out,
              src_v, dst_v, vals_v, dis_v, sl_v, pb_v, yf_v, u_full, b_v,
              u_sh, t_sh, w_sh, sem, sem_s, sem_d, sem_y):
    cid = lax.axis_index("c")
    sid = lax.axis_index("s")
    s0 = pl.multiple_of(sid * NPS, NPS)
    rs = pl.multiple_of(sid * ROWS, ROWS)
    nrows = jnp.where(sid == NS - 1, R_REAL - (NS - 1) * ROWS, ROWS)

    with jax.named_scope("stage"):
        pltpu.async_copy(ei3.at[0, pl.ds(rs, ROWS)], src_v, sem_s)
        pltpu.async_copy(ei3.at[1, pl.ds(rs, ROWS)], dst_v, sem_d)
        pltpu.async_copy(y2, yf_v, sem_y)
        pltpu.sync_copy(b16, b_v)

    with jax.named_scope("build_u0"):
        pltpu.sync_copy(degp.at[0, pl.ds(s0, NPS)], sl_v)
        pltpu.sync_copy(degp.at[1, pl.ds(s0, NPS)], pb_v)
        pltpu.make_async_copy(y2, yf_v, sem_y).wait()
        r0 = sid * (NPS // 128)
        for k in range(NPS // L):
            sl = pl.ds(k * L, L)
            dis = _rsqrt16(sl_v[sl] + pb_v[sl] + 1.0)
            yk = yf_v[r0 + k // 8, pl.ds((k % 8) * L, L)]
            dis_v[sl] = dis
            sl_v[sl] = dis * yk
        pltpu.sync_copy(sl_v, u_sh.at[pl.ds(s0, NPS)])
        pltpu.sync_copy(sl_v, t_sh.at[pl.ds(s0, NPS)])
    plsc.subcore_barrier()

    with jax.named_scope("hop1"):
        pltpu.make_async_copy(ei3.at[0, pl.ds(rs, ROWS)], src_v, sem_s).wait()
        pltpu.make_async_copy(ei3.at[1, pl.ds(rs, ROWS)], dst_v, sem_d).wait()
        pltpu.sync_copy(u_sh, u_full)

        @plsc.parallel_loop(0, nrows, unroll=4)
        def _(j):
            for c in range(128 // L):
                sl = pl.ds(c * L, L)
                vals_v[j, sl] = plsc.load_gather(u_full, [src_v[j, sl]])
            pltpu.async_copy(vals_v.at[j], t_sh.at[dst_v.at[j]], sem,
                             add=True)

        def drain1(j, carry):
            pltpu.make_async_copy(vals_v.at[j], t_sh.at[dst_v.at[j]],
                                  sem).wait()
            return carry

        lax.fori_loop(0, nrows, drain1, 0)
    plsc.subcore_barrier()

    with jax.named_scope("build_u1"):
        pltpu.sync_copy(t_sh.at[pl.ds(s0, NPS)], sl_v)
        for k in range(NPS // L):
            sl = pl.ds(k * L, L)
            dis = dis_v[sl]
            sl_v[sl] = dis * dis * sl_v[sl]
        pltpu.sync_copy(sl_v, u_sh.at[pl.ds(s0, NPS)])
        pltpu.sync_copy(sl_v, w_sh.at[pl.ds(s0, NPS)])
    plsc.subcore_barrier()

    with jax.named_scope("hop2"):
        pltpu.sync_copy(u_sh, u_full)

        @plsc.parallel_loop(0, nrows, unroll=4)
        def _(j):
            for c in range(128 // L):
                sl = pl.ds(c * L, L)
                vals_v[j, sl] = plsc.load_gather(u_full, [src_v[j, sl]])
            pltpu.async_copy(vals_v.at[j], w_sh.at[dst_v.at[j]], sem,
                             add=True)

        def drain2(j, carry):
            pltpu.make_async_copy(vals_v.at[j], w_sh.at[dst_v.at[j]],
                                  sem).wait()
            return carry

        lax.fori_loop(0, nrows, drain2, 0)
    plsc.subcore_barrier()

    with jax.named_scope("epilogue"):
        @pl.when(cid == 0)
        def _():
            pltpu.sync_copy(w_sh.at[pl.ds(s0, NPS)], sl_v)
            for k in range(NPS // L):
                sl = pl.ds(k * L, L)
                h = dis_v[sl] * sl_v[sl] + b_v[...]
                sl_v[sl] = h * h
            pltpu.sync_copy(sl_v, out.at[pl.ds(s0, NPS)])


@functools.lru_cache(maxsize=None)
def _deg_kernel():
    return functools.partial(
        pl.kernel,
        mesh=_mesh(),
        compiler_params=pltpu.CompilerParams(needs_layout_passes=False),
        out_type=[jax.ShapeDtypeStruct((NC, NPAD), jnp.float32)],
        scratch_types=[
            pltpu.VMEM((ROWS // 2, 128), jnp.int32),
            pltpu.VMEM((NPAD,), jnp.float32),
            pltpu.VMEM((NPS,), jnp.float32),
            pltpu.VMEM((128,), jnp.float32),
            pltpu.VMEM_SHARED((NPAD,), jnp.float32),
            pltpu.SemaphoreType.DMA,
            pltpu.SemaphoreType.DMA,
        ],
    )(_deg_body)


@functools.lru_cache(maxsize=None)
def _sc_kernel():
    return functools.partial(
        pl.kernel,
        mesh=_mesh(),
        compiler_params=pltpu.CompilerParams(needs_layout_passes=False),
        out_type=[jax.ShapeDtypeStruct((NPAD,), jnp.float32)],
        scratch_types=[
            pltpu.VMEM((ROWS, 128), jnp.int32),
            pltpu.VMEM((ROWS, 128), jnp.int32),
            pltpu.VMEM((ROWS, 128), jnp.float32),
            pltpu.VMEM((NPS,), jnp.float32),
            pltpu.VMEM((NPS,), jnp.float32),
            pltpu.VMEM((NPS,), jnp.float32),
            pltpu.VMEM((NB, 128), jnp.float32),
            pltpu.VMEM((NPAD,), jnp.float32),
            pltpu.VMEM((L,), jnp.float32),
            pltpu.VMEM_SHARED((NPAD,), jnp.float32),
            pltpu.VMEM_SHARED((NPAD,), jnp.float32),
            pltpu.VMEM_SHARED((NPAD,), jnp.float32),
            pltpu.SemaphoreType.DMA,
            pltpu.SemaphoreType.DMA,
            pltpu.SemaphoreType.DMA,
            pltpu.SemaphoreType.DMA,
        ],
    )(_sgc_body)


def kernel(x, edge_index, W, b):
    ei3 = jnp.pad(edge_index.astype(jnp.int32),
                  ((0, 0), (0, R_TOT * 128 - N_EDGES))).reshape(2, R_TOT, 128)
    xpad = jnp.pad(x.astype(jnp.float32), ((0, NPAD - N_NODES), (0, 0)))
    x3 = xpad.reshape(NB, 128, D_FEAT)
    wrow = W.astype(jnp.float32).reshape(1, D_FEAT)
    b16 = jnp.broadcast_to(b.astype(jnp.float32).reshape(1), (L,))
    zeros1 = jnp.zeros((NPAD,), jnp.float32)

    y2 = pl.pallas_call(
        _mv_body,
        out_shape=jax.ShapeDtypeStruct((NB, 128), jnp.float32),
    )(x3, wrow)

    degp = _deg_kernel()(ei3, zeros1)
    if isinstance(degp, (tuple, list)):
        (degp,) = degp
    o = _sc_kernel()(ei3, y2, degp, b16)
    if isinstance(o, (tuple, list)):
        (o,) = o
    return o.reshape(NPAD, 1)[:N_NODES]
